# baseline (speedup 1.0000x reference)
import functools

import jax
import jax.numpy as jnp
from jax import lax
from jax.experimental import pallas as pl
from jax.experimental.pallas import tpu as pltpu
from jax.experimental.pallas import tpu_sc as plsc

VOCAB = 128000
D_MODEL = 4096
NTOK = 16384

_info = plsc.get_sparse_core_info()
NC, NS = _info.num_cores, _info.num_subcores
NW = NC * NS
TPW = NTOK // NW
K = 8
ROUND = 3 * K
NROUNDS = TPW // ROUND


@functools.partial(
    pl.kernel,
    mesh=plsc.VectorSubcoreMesh(core_axis_name="c", subcore_axis_name="s"),
    out_type=jax.ShapeDtypeStruct((NTOK, D_MODEL), jnp.float32),
    scratch_types=[
        pltpu.VMEM((TPW,), jnp.int32),
        pltpu.VMEM((ROUND, D_MODEL), jnp.float32),
        pltpu.SemaphoreType.DMA,
        pltpu.SemaphoreType.DMA,
    ],
)
def _emb_lookup(ids_hbm, table_hbm, out_hbm, idx_v, buf, sem0, sem1):
    wid = lax.axis_index("s") * NC + lax.axis_index("c")
    base = wid * TPW
    pltpu.sync_copy(ids_hbm.at[pl.ds(base, TPW)], idx_v)

    def gather(tok, row0, nrows, sem):
        pltpu.async_copy(
            table_hbm.at[idx_v.at[pl.ds(tok, nrows)]],
            buf.at[pl.ds(row0, nrows)],
            sem,
        )

    def wait(row0, nrows, sem):
        pltpu.make_async_copy(
            table_hbm.at[pl.ds(0, nrows)], buf.at[pl.ds(row0, nrows)], sem
        ).wait()

    def write_out(row0, nrows, tok):
        pltpu.sync_copy(
            buf.at[pl.ds(row0, nrows)], out_hbm.at[pl.ds(base + tok, nrows)]
        )

    gather(0, 0, 2 * K, sem0)
    gather(2 * K, 2 * K, K, sem1)

    def round_body(r, carry):
        t = ROUND * r
        wait(0, 2 * K, sem0)
        write_out(0, 2 * K, t)

        @pl.when(r + 1 < NROUNDS)
        def _():
            gather(t + ROUND, 0, 2 * K, sem0)

        @pl.when(r + 1 == NROUNDS)
        def _():
            gather(t + ROUND, 0, K, sem0)

        wait(2 * K, K, sem1)
        write_out(2 * K, K, t + 2 * K)

        @pl.when(r + 1 < NROUNDS)
        def _():
            gather(t + ROUND + 2 * K, 2 * K, K, sem1)

        return carry

    lax.fori_loop(0, NROUNDS, round_body, 0)
    wait(0, K, sem0)
    write_out(0, K, TPW - K)


def kernel(input_ids, embed_weight):
    ids_flat = input_ids.reshape(NTOK).astype(jnp.int32)
    out = _emb_lookup(ids_flat, embed_weight)
    return out.reshape(input_ids.shape[0], input_ids.shape[1], D_MODEL)
